# baseline (speedup 1.0000x reference)
import jax
import jax.numpy as jnp
from jax.experimental import pallas as pl
from jax.experimental.pallas import tpu as pltpu

_BLK = 1024


def _make_body(B, blk):
    def _body(w_ref, o_ref, sem):
        i = pl.program_id(0)
        copies = [
            pltpu.make_async_copy(
                w_ref, o_ref.at[b, pl.ds(i * blk, blk), :], sem.at[b]
            )
            for b in range(B)
        ]
        for c in copies:
            c.start()
        for c in copies:
            c.wait()

    return _body


def kernel(x, W):
    B = x.shape[0]
    N, D = W.shape
    return pl.pallas_call(
        _make_body(B, _BLK),
        grid=(N // _BLK,),
        in_specs=[pl.BlockSpec((_BLK, D), lambda i: (i, 0))],
        out_specs=pl.BlockSpec(memory_space=pl.ANY),
        out_shape=jax.ShapeDtypeStruct((B, N, D), W.dtype),
        scratch_shapes=[pltpu.SemaphoreType.DMA((B,))],
    )(W)
